# Initial kernel scaffold; baseline (speedup 1.0000x reference)
import jax
import jax.numpy as jnp
from jax.experimental import pallas as pl


def kernel(z_x, adj_edge_index, W4, W5, W6):
    raise NotImplementedError("write your pallas kernel here")


# speedup vs baseline: 3.8398x; 3.8398x over previous
import functools

import jax
import jax.numpy as jnp
from jax import lax
from jax.experimental import pallas as pl
from jax.experimental.pallas import tpu as pltpu
from jax.experimental.pallas import tpu_sc as plsc

N = 10000
E = 320000
LATENT = 128
DOUT = 64

NC = 2
NS = 16
NW = NC * NS
EPW = E // NW
BATCH = 80
NITER = EPW // BATCH
RPS = 624
RTAIL = N - RPS * NS


def _make_spmm(d):
    mesh = plsc.VectorSubcoreMesh(core_axis_name="c", subcore_axis_name="s")

    @functools.partial(
        pl.kernel,
        out_type=jax.ShapeDtypeStruct((NC, N, d), jnp.float32),
        mesh=mesh,
        scratch_types=[
            pltpu.VMEM((BATCH,), jnp.int32),
            pltpu.VMEM((BATCH,), jnp.int32),
            pltpu.VMEM((BATCH, d), jnp.float32),
            pltpu.VMEM_SHARED((N, d), jnp.float32),
            pltpu.SemaphoreType.DMA,
        ],
    )
    def spmm(support, src, dst, zeros, out, src_v, dst_v, rows_v, acc, sem):
        cid = lax.axis_index("c")
        sid = lax.axis_index("s")
        wid = sid * NC + cid
        off = pl.multiple_of(sid * RPS, 8)
        pltpu.sync_copy(zeros.at[pl.ds(off, RPS)], acc.at[pl.ds(off, RPS)])

        @pl.when(sid == NS - 1)
        def _():
            pltpu.sync_copy(zeros.at[pl.ds(RPS * NS, RTAIL)],
                            acc.at[pl.ds(RPS * NS, RTAIL)])

        plsc.subcore_barrier()
        base0 = wid * EPW

        def body(it, carry):
            base = pl.multiple_of(base0 + it * BATCH, 8)
            pltpu.sync_copy(src.at[pl.ds(base, BATCH)], src_v)
            pltpu.sync_copy(dst.at[pl.ds(base, BATCH)], dst_v)
            pltpu.async_copy(support.at[src_v], rows_v, sem).wait()
            pltpu.sync_copy(rows_v, acc.at[dst_v], add=True)
            return carry

        lax.fori_loop(0, NITER, body, 0)
        plsc.subcore_barrier()
        pltpu.sync_copy(acc.at[pl.ds(off, RPS)], out.at[cid, pl.ds(off, RPS)])

        @pl.when(sid == NS - 1)
        def _():
            pltpu.sync_copy(acc.at[pl.ds(RPS * NS, RTAIL)],
                            out.at[cid, pl.ds(RPS * NS, RTAIL)])

    return spmm


_spmm128 = _make_spmm(LATENT)

BM = 1000
BDI = 1000
BDJ = 1280


def _tanh_mm(x, w):
    din, dout = w.shape

    def body(x_ref, w_ref, o_ref):
        o_ref[...] = jnp.tanh(
            jnp.dot(x_ref[...], w_ref[...], preferred_element_type=jnp.float32))

    return pl.pallas_call(
        body,
        grid=(N // BM,),
        in_specs=[pl.BlockSpec((BM, din), lambda i: (i, 0)),
                  pl.BlockSpec((din, dout), lambda i: (0, 0))],
        out_specs=pl.BlockSpec((BM, dout), lambda i: (i, 0)),
        out_shape=jax.ShapeDtypeStruct((N, dout), jnp.float32),
    )(x, w)


def _tanh_mm_partials(p, w):
    din, dout = w.shape

    def body(p_ref, w_ref, o_ref):
        x = p_ref[0] + p_ref[1]
        o_ref[...] = jnp.tanh(
            jnp.dot(x, w_ref[...], preferred_element_type=jnp.float32))

    return pl.pallas_call(
        body,
        grid=(N // BM,),
        in_specs=[pl.BlockSpec((2, BM, din), lambda i: (0, i, 0)),
                  pl.BlockSpec((din, dout), lambda i: (0, 0))],
        out_specs=pl.BlockSpec((BM, dout), lambda i: (i, 0)),
        out_shape=jax.ShapeDtypeStruct((N, dout), jnp.float32),
    )(p, w)


def _decode(p):
    def body(a_ref, b_ref, x_ref, adj_ref):
        xi = a_ref[0] + a_ref[1]
        xj = b_ref[0] + b_ref[1]
        x_ref[...] = xi[:, :DOUT]
        logits = lax.dot_general(xi, xj, (((1,), (1,)), ((), ())),
                                 preferred_element_type=jnp.float32)
        adj_ref[...] = jax.nn.sigmoid(logits)

    return pl.pallas_call(
        body,
        grid=(N // BDI, (N + BDJ - 1) // BDJ),
        in_specs=[pl.BlockSpec((2, BDI, LATENT), lambda i, j: (0, i, 0)),
                  pl.BlockSpec((2, BDJ, LATENT), lambda i, j: (0, j, 0))],
        out_specs=[pl.BlockSpec((BDI, DOUT), lambda i, j: (i, 0)),
                   pl.BlockSpec((BDI, BDJ), lambda i, j: (i, j))],
        out_shape=[jax.ShapeDtypeStruct((N, DOUT), jnp.float32),
                   jax.ShapeDtypeStruct((N, N), jnp.float32)],
    )(p, p)


def kernel(z_x, adj_edge_index, W4, W5, W6):
    dst = adj_edge_index[0]
    src = adj_edge_index[1]
    z128 = jnp.zeros((N, LATENT), jnp.float32)
    w6p = jnp.pad(W6, ((0, 0), (0, LATENT - DOUT)))

    s = _tanh_mm(z_x, W4)
    p = _spmm128(s, src, dst, z128)
    s = _tanh_mm_partials(p, W5)
    p = _spmm128(s, src, dst, z128)
    s = _tanh_mm_partials(p, w6p)
    p = _spmm128(s, src, dst, z128)
    x_hat, adj_hat = _decode(p)
    return (x_hat, adj_hat)
